# baseline (speedup 1.0000x reference)
import functools

import jax
import jax.numpy as jnp
from jax import lax
from jax.experimental import pallas as pl
from jax.experimental.pallas import tpu as pltpu
from jax.experimental.pallas import tpu_sc as plsc

NBUF = 2


@functools.lru_cache(maxsize=None)
def _build_gather(batch: int, fields: int, dim: int, nbuf: int):
    mesh = plsc.VectorSubcoreMesh(core_axis_name="c", subcore_axis_name="s")
    n_workers = mesh.num_cores * mesh.num_subcores
    assert batch % n_workers == 0
    rows = batch // n_workers

    @functools.partial(
        pl.kernel,
        out_type=jax.ShapeDtypeStruct((batch, fields, dim), jnp.float32),
        mesh=mesh,
        scratch_types=[
            [pltpu.VMEM((rows,), jnp.int32) for _ in range(nbuf)],
            [pltpu.VMEM((rows, dim), jnp.float32) for _ in range(nbuf)],
            [pltpu.SemaphoreType.DMA for _ in range(nbuf)],
            [pltpu.SemaphoreType.DMA for _ in range(nbuf)],
            [pltpu.SemaphoreType.DMA for _ in range(nbuf)],
        ],
        compiler_params=pltpu.CompilerParams(use_tc_tiling_on_sc=False),
    )
    def gather_kernel(idx_hbm, table_hbm, out_hbm,
                      idx_v, rows_v, isem, gsem, ssem):
        wid = lax.axis_index("s") * mesh.num_cores + lax.axis_index("c")
        base = wid * rows

        def stage(b, f):
            pltpu.async_copy(idx_hbm.at[f, pl.ds(base, rows)],
                             idx_v[b], isem[b]).wait()
            pltpu.async_copy(table_hbm.at[idx_v[b]], rows_v[b], gsem[b])

        def out_slice(f):
            return out_hbm.at[pl.ds(base, rows), f, :]

        for b in range(nbuf):
            stage(b, b)

        def body(grp, carry):
            f0 = grp * nbuf
            for b in range(nbuf):
                f = f0 + b
                pltpu.make_async_copy(table_hbm.at[idx_v[b]], rows_v[b],
                                      gsem[b]).wait()
                pltpu.async_copy(rows_v[b], out_slice(f), ssem[b])

                @pl.when(f + nbuf < fields)
                def _():
                    pltpu.make_async_copy(rows_v[b], out_slice(f),
                                          ssem[b]).wait()
                    stage(b, f + nbuf)

            return carry

        lax.fori_loop(0, fields // nbuf, body, 0)

        rem = fields % nbuf
        for b in range(rem):
            f = (fields // nbuf) * nbuf + b
            pltpu.make_async_copy(table_hbm.at[idx_v[b]], rows_v[b],
                                  gsem[b]).wait()
            pltpu.async_copy(rows_v[b], out_slice(f), ssem[b])
        for b in range(nbuf):
            f = fields - nbuf + b
            pltpu.make_async_copy(rows_v[b], out_slice(f), ssem[b]).wait()

    return gather_kernel


def kernel(input, weight):
    b, f = input.shape
    return _build_gather(b, f, weight.shape[1], NBUF)(
        input.astype(jnp.int32).T, weight)
